# baseline (speedup 1.0000x reference)
import functools

import jax
import jax.numpy as jnp
from jax import lax
from jax.experimental import pallas as pl
from jax.experimental.pallas import tpu as pltpu
from jax.experimental.pallas import tpu_sc as plsc

D = 64
SEQ = 200
BATCH = 4096
ROWS = BATCH * SEQ
NC, NS = 2, 16
NW = NC * NS
ROWS_PER_W = ROWS // NW
CHUNK = 400
N_CHUNKS = ROWS_PER_W // CHUNK
GS = (128, 128, 128, 16)
SCALE = 8.0


def _body(x_hbm, tab_hbm, pos_hbm, out_hbm, xv, idx2, slab, outb, pos2, sem):
    wid = lax.axis_index("s") * NC + lax.axis_index("c")
    base_w = wid * ROWS_PER_W
    pltpu.sync_copy(pos_hbm, pos2.at[pl.ds(0, SEQ * D)])
    pltpu.sync_copy(pos_hbm, pos2.at[pl.ds(SEQ * D, SEQ * D)])

    def chunk_body(c, carry):
        base = base_w + c * CHUNK
        pltpu.sync_copy(x_hbm.at[pl.ds(base, CHUNK)], xv)
        for k in range(CHUNK // 16):
            sl = pl.ds(k * 16, 16)
            idx2[sl] = lax.shift_right_logical(xv[sl], 1)
        off = 0
        copies = []
        for g in GS:
            copies.append(
                pltpu.async_copy(
                    tab_hbm.at[idx2.at[pl.ds(off, g)]],
                    slab.at[pl.ds(off, g)],
                    sem,
                )
            )
            off += g
        for cp in copies:
            cp.wait()

        def grp(g, carry2):
            gsl = pl.ds(g * 16, 16)
            offv = (xv[gsl] & 1) * 64
            for i in range(16):
                r = g * 16 + i
                o = offv[i]
                for q in range(D // 16):
                    outb[r, pl.ds(q * 16, 16)] = (
                        slab[r, pl.ds(o + q * 16, 16)] * SCALE
                        + pos2[pl.ds(r * D + q * 16, 16)]
                    )
            return carry2

        lax.fori_loop(0, CHUNK // 16, grp, 0)
        pltpu.sync_copy(outb, out_hbm.at[pl.ds(base, CHUNK)])
        return carry

    lax.fori_loop(0, N_CHUNKS, chunk_body, 0)


@jax.jit
def kernel(x, token_table, pos_table):
    x_flat = x.reshape(-1).astype(jnp.int32)
    tab2 = token_table.reshape(ROWS // ROWS * 500000, 128)
    pos_flat = pos_table.reshape(-1)
    mesh = plsc.VectorSubcoreMesh(core_axis_name="c", subcore_axis_name="s")
    run = pl.kernel(
        _body,
        mesh=mesh,
        compiler_params=pltpu.CompilerParams(use_tc_tiling_on_sc=True),
        out_type=jax.ShapeDtypeStruct((ROWS, D), jnp.float32),
        scratch_types=[
            pltpu.VMEM((CHUNK,), jnp.int32),
            pltpu.VMEM((CHUNK,), jnp.int32),
            pltpu.VMEM((CHUNK, 128), jnp.float32),
            pltpu.VMEM((CHUNK, D), jnp.float32),
            pltpu.VMEM((2 * SEQ * D,), jnp.float32),
            pltpu.SemaphoreType.DMA,
        ],
    )
    out = run(x_flat, tab2, pos_flat)
    return out.reshape(BATCH, SEQ, D)
